# baseline (speedup 1.0000x reference)
import functools

import jax
import jax.numpy as jnp
from jax import lax
from jax.experimental import pallas as pl
from jax.experimental.pallas import tpu as pltpu
from jax.experimental.pallas import tpu_sc as plsc

N = 50000
E = 800000
F_IN = 128
HID = 64
HEADS = 4
OUTC = 16
G = 64
EMB = 128

BN = 2000
NB = N // BN
CH = 128
NCHUNK = E // CH
ZCH = 2000
NZ = N // ZCH

_HI = jax.lax.Precision.HIGHEST


def _dot(a, b, dims):
    return lax.dot_general(a, b, (dims, ((), ())), precision=_HI,
                           preferred_element_type=jnp.float32)


@functools.cache
def _sc_mesh():
    return plsc.VectorSubcoreMesh(core_axis_name="c", subcore_axis_name="s",
                                  num_cores=2, num_subcores=16)


def _prep_body(x_ref, w_ref, asx_ref, adx_ref, hh_ref, aS_ref, aD_ref,
               ex_ref):
    hhfull = _dot(x_ref[...], w_ref[...], ((1,), (1,)))
    for t in range(2 * HEADS):
        hh_ref[t] = hhfull[:, t * 8:(t + 1) * 8]
    a_s = _dot(asx_ref[...], hhfull, ((1,), (1,)))
    a_d = _dot(adx_ref[...], hhfull, ((1,), (1,)))
    aS_ref[0] = a_s
    aD_ref[0] = a_d
    al = a_s + a_d
    ex_ref[0] = jnp.exp(jnp.where(al > 0, al, 0.2 * al))


def _prep(x, W, asx, adx):
    return pl.pallas_call(
        _prep_body,
        grid=(NB,),
        in_specs=[
            pl.BlockSpec((BN, F_IN), lambda i: (i, 0)),
            pl.BlockSpec((HID, F_IN), lambda i: (0, 0)),
            pl.BlockSpec((HEADS, HID), lambda i: (0, 0)),
            pl.BlockSpec((HEADS, HID), lambda i: (0, 0)),
        ],
        out_specs=[
            pl.BlockSpec((2 * HEADS, BN, 8), lambda i: (0, i, 0)),
            pl.BlockSpec((1, HEADS, BN), lambda i: (i, 0, 0)),
            pl.BlockSpec((1, HEADS, BN), lambda i: (i, 0, 0)),
            pl.BlockSpec((1, HEADS, BN), lambda i: (i, 0, 0)),
        ],
        out_shape=[
            jax.ShapeDtypeStruct((2 * HEADS, N, 8), jnp.float32),
            jax.ShapeDtypeStruct((NB, HEADS, BN), jnp.float32),
            jax.ShapeDtypeStruct((NB, HEADS, BN), jnp.float32),
            jax.ShapeDtypeStruct((NB, HEADS, BN), jnp.float32),
        ],
    )(x, W, asx, adx)


def _combine_cols(num_ref, den_ref, exs_ref, hh_ref, bngb_ref):
    cols = []
    for t in range(2 * HEADS):
        h = t // 2
        numh = num_ref[t]
        hhh = hh_ref[t]
        exh = exs_ref[0, h]
        dtot = den_ref[0, h] + exh + 1e-16
        cols.append((numh + exh[:, None] * hhh) / dtot[:, None])
    hnew = jnp.concatenate(cols, axis=1)
    bb = bngb_ref[0][None, :]
    gg = bngb_ref[1][None, :]
    be = bngb_ref[2][None, :]
    return jnp.maximum((hnew + bb) * gg + be, 0.0)


def _combine_body(num_ref, den_ref, exs_ref, hh_ref, bngb_ref, out_ref):
    hnew = _combine_cols(num_ref, den_ref, exs_ref, hh_ref, bngb_ref)
    out_ref[...] = jnp.concatenate(
        [hnew, jnp.zeros((BN, F_IN - HID), jnp.float32)], axis=1)


def _combine_in_specs():
    return [
        pl.BlockSpec((2 * HEADS, BN, 8), lambda i: (0, i, 0)),
        pl.BlockSpec((1, HEADS, BN), lambda i: (i, 0, 0)),
        pl.BlockSpec((1, HEADS, BN), lambda i: (i, 0, 0)),
        pl.BlockSpec((2 * HEADS, BN, 8), lambda i: (0, i, 0)),
        pl.BlockSpec((3, HID), lambda i: (0, 0)),
    ]


def _combine(num, den, exs, hh, bngb):
    return pl.pallas_call(
        _combine_body,
        grid=(NB,),
        in_specs=_combine_in_specs(),
        out_specs=pl.BlockSpec((BN, F_IN), lambda i: (i, 0)),
        out_shape=jax.ShapeDtypeStruct((N, F_IN), jnp.float32),
    )(num, den, exs, hh, bngb)


def _logits_body(h_ref, pw1_ref, pb1_ref, pw2_ref, pb2_ref, frag_ref,
                 lg_ref, bm_ref):
    h3 = h_ref[...][:, :HID]
    t = jnp.maximum(_dot(h3, pw1_ref[...], ((1,), (1,))) + pb1_ref[0][None, :],
                    0.0)
    lg = _dot(t, pw2_ref[...], ((1,), (1,)))
    lgv = lg[:, 0] + pb2_ref[0, 0]
    lg_ref[...] = lgv.reshape(1, 1, BN)
    labb = frag_ref[0, 0, :]
    m0 = jnp.max(jnp.where(labb == 0, lgv, -1e30))
    m1 = jnp.max(jnp.where(labb == 1, lgv, -1e30))
    lane = lax.broadcasted_iota(jnp.int32, (1, 1, 128), 2)
    bm_ref[...] = jnp.where(lane == 0, m0, jnp.where(lane == 1, m1, -1e30))


def _logits(hpad, pw1, pb1, pw2, pb2, frag3d):
    return pl.pallas_call(
        _logits_body,
        grid=(NB,),
        in_specs=[
            pl.BlockSpec((BN, F_IN), lambda i: (i, 0)),
            pl.BlockSpec((HID // 2, HID), lambda i: (0, 0)),
            pl.BlockSpec((1, HID // 2), lambda i: (0, 0)),
            pl.BlockSpec((8, HID // 2), lambda i: (0, 0)),
            pl.BlockSpec((1, 1), lambda i: (0, 0)),
            pl.BlockSpec((1, 1, BN), lambda i: (i, 0, 0)),
        ],
        out_specs=[
            pl.BlockSpec((1, 1, BN), lambda i: (i, 0, 0)),
            pl.BlockSpec((1, 1, 128), lambda i: (i, 0, 0)),
        ],
        out_shape=[
            jax.ShapeDtypeStruct((NB, 1, BN), jnp.float32),
            jax.ShapeDtypeStruct((NB, 1, 128), jnp.float32),
        ],
    )(hpad, pw1, pb1, pw2, pb2, frag3d)


def _pool_body(h_ref, lg_ref, batch_ref, frag_ref, bm_ref, f1_ref, f2_ref,
               F_acc, s_acc):
    i = pl.program_id(0)

    @pl.when(i == 0)
    def _init():
        F_acc[...] = jnp.zeros((2, G, HID), jnp.float32)
        s_acc[0] = 0.0
        s_acc[1] = 0.0

    mv = jnp.max(bm_ref[...], axis=(0, 1))
    m0 = mv[0]
    m1 = mv[1]
    lgv = lg_ref[0, 0, :]
    labb = frag_ref[0, 0, :]
    bb = batch_ref[0, 0, :]
    e0 = jnp.where(labb == 0, jnp.exp(lgv - m0), 0.0)
    e1 = jnp.where(labb == 1, jnp.exp(lgv - m1), 0.0)
    gid = lax.broadcasted_iota(jnp.int32, (BN, G), 1)
    oh = (bb[:, None] == gid).astype(jnp.float32)
    h3 = h_ref[...][:, :HID]
    A0 = oh * e0[:, None]
    A1 = oh * e1[:, None]
    F_acc[0] += _dot(A0, h3, ((0,), (0,)))
    F_acc[1] += _dot(A1, h3, ((0,), (0,)))
    s_acc[0] += jnp.sum(e0)
    s_acc[1] += jnp.sum(e1)
    f1_ref[...] = F_acc[0] / s_acc[0]
    f2_ref[...] = F_acc[1] / s_acc[1]


def _pool(hpad, lg3d, batch3d, frag3d, bm):
    return pl.pallas_call(
        _pool_body,
        grid=(NB,),
        in_specs=[
            pl.BlockSpec((BN, F_IN), lambda i: (i, 0)),
            pl.BlockSpec((1, 1, BN), lambda i: (i, 0, 0)),
            pl.BlockSpec((1, 1, BN), lambda i: (i, 0, 0)),
            pl.BlockSpec((1, 1, BN), lambda i: (i, 0, 0)),
            pl.BlockSpec((NB, 1, 128), lambda i: (0, 0, 0)),
        ],
        out_specs=[
            pl.BlockSpec((G, HID), lambda i: (0, 0)),
            pl.BlockSpec((G, HID), lambda i: (0, 0)),
        ],
        out_shape=[
            jax.ShapeDtypeStruct((G, HID), jnp.float32),
            jax.ShapeDtypeStruct((G, HID), jnp.float32),
        ],
        scratch_shapes=[
            pltpu.VMEM((2, G, HID), jnp.float32),
            pltpu.SMEM((2,), jnp.float32),
        ],
        compiler_params=pltpu.CompilerParams(
            dimension_semantics=("arbitrary",)),
    )(hpad, lg3d, batch3d, frag3d, bm)


def _heads_body(f1_ref, f2_ref, iw1_ref, ib1_ref, iw2_ref, ib2_ref,
                cw1_ref, cb1_ref, cw2_ref, cb2_ref, ln_ref,
                sw1_ref, sb1_ref, sw2_ref, sb2_ref, hw_ref, hb_ref,
                c1_ref, c2_ref, sm_ref):
    f1 = f1_ref[...]
    f2 = f2_ref[...]

    def contrast(f):
        t = jnp.maximum(_dot(f, cw1_ref[...], ((1,), (1,))) + cb1_ref[0][None, :], 0.0)
        p = _dot(t, cw2_ref[...], ((1,), (1,))) + cb2_ref[0][None, :]
        mu = jnp.mean(p, axis=1, keepdims=True)
        var = jnp.mean((p - mu) ** 2, axis=1, keepdims=True)
        p = (p - mu) / jnp.sqrt(var + 1e-5) * ln_ref[0][None, :] + ln_ref[1][None, :]
        nrm = jnp.sqrt(jnp.sum(p * p, axis=1, keepdims=True))
        return p / jnp.maximum(nrm, 1e-12)

    c1_ref[...] = contrast(f1)
    c2_ref[...] = contrast(f2)

    comb = jnp.concatenate([f1, f2], axis=1)
    t = jnp.maximum(_dot(comb, iw1_ref[...], ((1,), (1,))) + ib1_ref[0][None, :], 0.0)
    inter = jnp.maximum(_dot(t, iw2_ref[...], ((1,), (1,))) + ib2_ref[0][None, :], 0.0)
    t = jnp.maximum(_dot(inter, sw1_ref[...], ((1,), (1,))) + sb1_ref[0][None, :], 0.0)
    sh = jnp.maximum(_dot(t, sw2_ref[...], ((1,), (1,))) + sb2_ref[0][None, :], 0.0)
    v = _dot(sh, hw_ref[...], ((1,), (1,))) + hb_ref[0][None, :]
    lane = lax.broadcasted_iota(jnp.int32, (G, 8), 1)
    sm_ref[...] = jnp.where(lane == 1, jax.nn.sigmoid(v), v)


def _heads(f1, f2, pp):
    ins = [f1, f2, pp['iw1'], pp['ib1'], pp['iw2'], pp['ib2'],
           pp['cw1'], pp['cb1'], pp['cw2'], pp['cb2'], pp['ln'],
           pp['sw1'], pp['sb1'], pp['sw2'], pp['sb2'], pp['hw'], pp['hb']]
    return pl.pallas_call(
        _heads_body,
        in_specs=[pl.BlockSpec(a.shape, lambda: tuple(0 for _ in a.shape))
                  for a in ins],
        out_specs=[
            pl.BlockSpec((G, EMB), lambda: (0, 0)),
            pl.BlockSpec((G, EMB), lambda: (0, 0)),
            pl.BlockSpec((G, 8), lambda: (0, 0)),
        ],
        out_shape=[
            jax.ShapeDtypeStruct((G, EMB), jnp.float32),
            jax.ShapeDtypeStruct((G, EMB), jnp.float32),
            jax.ShapeDtypeStruct((G, 8), jnp.float32),
        ],
    )(*ins)


def _edge_body(hh_ref, aS_ref, aD_ref, src_ref, dst_ref, zr_ref,
               zd_ref, num_ref, den_ref,
               colS, colD, srcv0, srcv1, dstv0, dstv1,
               exv0, exv1, rows0, rows1, num_acc, den_acc,
               semG0, semG1):
    srcvs = [srcv0, srcv1]
    dstvs = [dstv0, dstv1]
    exvs = [exv0, exv1]
    rowss = [rows0, rows1]
    semGs = [semG0, semG1]


    c = lax.axis_index("c")
    s = lax.axis_index("s")

    nzr = NZ - (NZ // 16) * 16
    nz = jnp.where(s < nzr, NZ // 16 + 1, NZ // 16)
    ncr = NCHUNK - (NCHUNK // 16) * 16
    nchunks = jnp.where(s < ncr, NCHUNK // 16 + 1, NCHUNK // 16)

    iota = lax.iota(jnp.int32, 16)
    rowoff = iota // 8
    coloff = iota % 8

    def phase_body(p, _):
        hs = p // 2
        half = p % 2
        h = 2 * hs + c
        t = 2 * h + half

        def zero_body(j, _):
            base = (s + 16 * j) * ZCH
            pltpu.sync_copy(zr_ref, num_acc.at[pl.ds(base, ZCH)])

            @pl.when(half == 0)
            def _():
                pltpu.sync_copy(zd_ref, den_acc.at[pl.ds(base, ZCH)])
            return 0

        lax.fori_loop(0, nz, zero_body, 0)

        @pl.when(half == 0)
        def _():
            def col_body(j, _):
                pltpu.sync_copy(aS_ref.at[j, h], colS.at[pl.ds(j * BN, BN)])
                pltpu.sync_copy(aD_ref.at[j, h], colD.at[pl.ds(j * BN, BN)])
                return 0

            lax.fori_loop(0, NB, col_body, 0)

        plsc.subcore_barrier()

        def issue_idx(k, b):
            eb = (s + k * 16) * CH
            pltpu.sync_copy(src_ref.at[pl.ds(eb, CH)], srcvs[b])
            pltpu.sync_copy(dst_ref.at[pl.ds(eb, CH)], dstvs[b])

        def wait_idx(k, b):
            return

        issue_idx(0, 0)

        def group_body(g, _):
            for b in range(2):
                tt = g * 2 + b
                p = 1 - b
                nx = 1 - b

                @pl.when(tt <= nchunks - 1)
                def _():
                    wait_idx(tt, b)
                    for i in range(CH // 16):
                        sidx = srcvs[b][pl.ds(i * 16, 16)]
                        didx = dstvs[b][pl.ds(i * 16, 16)]
                        av = plsc.load_gather(colS, [sidx])
                        bv = plsc.load_gather(colD, [didx])
                        al = av + bv
                        al = jnp.where(al > 0, al, 0.2 * al)
                        exvs[b][pl.ds(i * 16, 16)] = jnp.exp(al)
                        srcvs[b][pl.ds(i * 16, 16)] = sidx + t * N
                    pltpu.async_copy(hh_ref.at[srcvs[b]], rowss[b], semGs[b])

                @pl.when((tt >= 1) & (tt <= nchunks))
                def _():
                    pltpu.make_async_copy(hh_ref.at[srcvs[p]], rowss[p],
                                          semGs[p]).wait()
                    for i in range(CH // 16):
                        ev = exvs[p][pl.ds(i * 16, 16)]
                        for gg in range(8):
                            e0 = ev[2 * gg]
                            e1 = ev[2 * gg + 1]
                            ridx = rowoff + (i * 16 + 2 * gg)
                            v = plsc.load_gather(rowss[p], [ridx, coloff])
                            sc = jnp.where(iota < 8, e0, e1)
                            plsc.store_scatter(rowss[p], [ridx, coloff],
                                               v * sc)
                    pltpu.sync_copy(rowss[p], num_acc.at[dstvs[p]], add=True)

                    @pl.when(half == 0)
                    def _():
                        pltpu.sync_copy(exvs[p], den_acc.at[dstvs[p]],
                                        add=True)

                @pl.when(tt + 1 <= nchunks - 1)
                def _():
                    issue_idx(tt + 1, nx)
            return 0

        ngroups = (nchunks + 2) // 2
        lax.fori_loop(0, ngroups, group_body, 0)
        plsc.subcore_barrier()

        def out_body(j, _):
            jb = s + 16 * j
            base = jb * ZCH
            pltpu.sync_copy(num_acc.at[pl.ds(base, ZCH)],
                            num_ref.at[pl.ds(t * N + base, ZCH)])

            @pl.when(half == 0)
            def _():
                pltpu.sync_copy(den_acc.at[pl.ds(base, ZCH)],
                                den_ref.at[jb, h])
            return 0

        lax.fori_loop(0, nz, out_body, 0)
        plsc.subcore_barrier()
        return 0

    lax.fori_loop(0, 4, phase_body, 0)


def _edge_pass(hh_flat, aS_flat, aD_flat, src, dst, zrows, zden):
    kern = pl.kernel(
        _edge_body,
        out_type=(
            jax.ShapeDtypeStruct((2 * HEADS * N, 8), jnp.float32),
            jax.ShapeDtypeStruct((NB, HEADS, BN), jnp.float32),
        ),
        mesh=_sc_mesh(),
        scratch_types=(
            [pltpu.VMEM((N,), jnp.float32)] * 2
            + [pltpu.VMEM((CH,), jnp.int32)] * 4
            + [pltpu.VMEM((CH,), jnp.float32)] * 2
            + [pltpu.VMEM((CH, 8), jnp.float32)] * 2
            + [pltpu.VMEM_SHARED((N, 8), jnp.float32),
               pltpu.VMEM_SHARED((N,), jnp.float32)]
            + [pltpu.SemaphoreType.DMA] * 2
        ),
        compiler_params=pltpu.CompilerParams(needs_layout_passes=False,
                                             use_tc_tiling_on_sc=False),
    )
    return kern(hh_flat, aS_flat, aD_flat, src, dst, zrows, zden)


def kernel(x, edge_index, batch, fragment_labels, params):
    src = edge_index[0]
    dst = edge_index[1]
    zrows = jnp.zeros((ZCH, 8), jnp.float32)
    zden = jnp.zeros((ZCH,), jnp.float32)

    gat = params['gat']
    bn = params['bn']

    def axp(a):
        eye = jnp.eye(HEADS, dtype=jnp.float32)
        return (eye[:, :, None] * a[:, None, :]).reshape(HEADS, HID)

    def padw(w):
        return jnp.concatenate(
            [w, jnp.zeros((HID, F_IN - w.shape[1]), jnp.float32)], axis=1) \
            if w.shape[1] < F_IN else w

    Wp = jnp.stack([padw(gat[l]['W']) for l in range(3)])
    asxp = jnp.stack([axp(gat[l]['as']) for l in range(3)])
    adxp = jnp.stack([axp(gat[l]['ad']) for l in range(3)])
    bngbp = jnp.stack([jnp.stack([gat[l]['b'], bn[l]['g'], bn[l]['b']])
                       for l in range(3)])
    def layer_body(h, xs):
        W, asx, adx, bngb = xs
        hh, aS, aD, exs = _prep(h, W, asx, adx)
        hf = hh.reshape(2 * HEADS * N, 8)
        num, den = _edge_pass(hf, aS, aD, src, dst, zrows, zden)
        num = num.reshape(2 * HEADS, N, 8)
        hnew = _combine(num, den, exs, hh, bngb)
        return hnew, None

    hpad, _ = lax.scan(layer_body, x, (Wp, asxp, adxp, bngbp))

    pool = params['pool']
    frag3d = fragment_labels.reshape(NB, 1, BN)
    batch3d = batch.reshape(NB, 1, BN)
    pw2p = jnp.concatenate(
        [pool['w2'], jnp.zeros((7, HID // 2), jnp.float32)], axis=0)
    lg3d, bm = _logits(hpad, pool['w1'], pool['b1'].reshape(1, -1),
                       pw2p, pool['b2'].reshape(1, 1), frag3d)
    f1, f2 = _pool(hpad, lg3d, batch3d, frag3d, bm)

    it = params['inter']
    cp = params['contr']
    mt = params['mt']
    hw = jnp.concatenate([mt['cw'], mt['rw'], mt['mw'],
                          jnp.zeros((4, HID // 8), jnp.float32)], axis=0)
    hb = jnp.concatenate([mt['cb'], mt['rb'], mt['mb'],
                          jnp.zeros((4,), jnp.float32)]).reshape(1, 8)
    pp = {
        'iw1': it['w1'], 'ib1': it['b1'].reshape(1, -1),
        'iw2': it['w2'], 'ib2': it['b2'].reshape(1, -1),
        'cw1': cp['w1'], 'cb1': cp['b1'].reshape(1, -1),
        'cw2': cp['w2'], 'cb2': cp['b2'].reshape(1, -1),
        'ln': jnp.stack([cp['lng'], cp['lnb']]),
        'sw1': mt['sw1'], 'sb1': mt['sb1'].reshape(1, -1),
        'sw2': mt['sw2'], 'sb2': mt['sb2'].reshape(1, -1),
        'hw': hw, 'hb': hb,
    }
    c1, c2, sm = _heads(f1, f2, pp)
    cls = sm[:, 0:1]
    reg = sm[:, 1:2]
    mc = sm[:, 2:4]
    return (c1, c2, cls, reg, mc)


# speedup vs baseline: 1.2854x; 1.2854x over previous
import functools

import jax
import jax.numpy as jnp
from jax import lax
from jax.experimental import pallas as pl
from jax.experimental.pallas import tpu as pltpu
from jax.experimental.pallas import tpu_sc as plsc

N = 50000
E = 800000
F_IN = 128
HID = 64
HEADS = 4
OUTC = 16
G = 64
EMB = 128

BN = 2000
NB = N // BN
CH = 128
NCHUNK = E // CH
ZCH = 2000
NZ = N // ZCH

_HI = jax.lax.Precision.HIGHEST


def _dot(a, b, dims):
    return lax.dot_general(a, b, (dims, ((), ())), precision=_HI,
                           preferred_element_type=jnp.float32)


@functools.cache
def _sc_mesh():
    return plsc.VectorSubcoreMesh(core_axis_name="c", subcore_axis_name="s",
                                  num_cores=2, num_subcores=16)


def _prep_body(x_ref, w_ref, asx_ref, adx_ref, hh_ref, aS_ref, aD_ref,
               ex_ref):
    hhfull = _dot(x_ref[...], w_ref[...], ((1,), (1,)))
    for t in range(2 * HEADS):
        hh_ref[t] = hhfull[:, t * 8:(t + 1) * 8]
    a_s = _dot(asx_ref[...], hhfull, ((1,), (1,)))
    a_d = _dot(adx_ref[...], hhfull, ((1,), (1,)))
    aS_ref[0] = a_s
    aD_ref[0] = a_d
    al = a_s + a_d
    ex_ref[0] = jnp.exp(jnp.where(al > 0, al, 0.2 * al))


def _prep(x, W, asx, adx):
    return pl.pallas_call(
        _prep_body,
        grid=(NB,),
        in_specs=[
            pl.BlockSpec((BN, F_IN), lambda i: (i, 0)),
            pl.BlockSpec((HID, F_IN), lambda i: (0, 0)),
            pl.BlockSpec((HEADS, HID), lambda i: (0, 0)),
            pl.BlockSpec((HEADS, HID), lambda i: (0, 0)),
        ],
        out_specs=[
            pl.BlockSpec((2 * HEADS, BN, 8), lambda i: (0, i, 0)),
            pl.BlockSpec((1, HEADS, BN), lambda i: (i, 0, 0)),
            pl.BlockSpec((1, HEADS, BN), lambda i: (i, 0, 0)),
            pl.BlockSpec((1, HEADS, BN), lambda i: (i, 0, 0)),
        ],
        out_shape=[
            jax.ShapeDtypeStruct((2 * HEADS, N, 8), jnp.float32),
            jax.ShapeDtypeStruct((NB, HEADS, BN), jnp.float32),
            jax.ShapeDtypeStruct((NB, HEADS, BN), jnp.float32),
            jax.ShapeDtypeStruct((NB, HEADS, BN), jnp.float32),
        ],
    )(x, W, asx, adx)


def _combine_cols(num_ref, den_ref, exs_ref, hh_ref, bngb_ref):
    cols = []
    for t in range(2 * HEADS):
        h = t // 2
        numh = num_ref[t]
        hhh = hh_ref[t]
        exh = exs_ref[0, h]
        dtot = den_ref[0, h] + exh + 1e-16
        cols.append((numh + exh[:, None] * hhh) / dtot[:, None])
    hnew = jnp.concatenate(cols, axis=1)
    bb = bngb_ref[0][None, :]
    gg = bngb_ref[1][None, :]
    be = bngb_ref[2][None, :]
    return jnp.maximum((hnew + bb) * gg + be, 0.0)


def _combine_body(num_ref, den_ref, exs_ref, hh_ref, bngb_ref, out_ref):
    hnew = _combine_cols(num_ref, den_ref, exs_ref, hh_ref, bngb_ref)
    out_ref[...] = jnp.concatenate(
        [hnew, jnp.zeros((BN, F_IN - HID), jnp.float32)], axis=1)


def _combine_in_specs():
    return [
        pl.BlockSpec((2 * HEADS, BN, 8), lambda i: (0, i, 0)),
        pl.BlockSpec((1, HEADS, BN), lambda i: (i, 0, 0)),
        pl.BlockSpec((1, HEADS, BN), lambda i: (i, 0, 0)),
        pl.BlockSpec((2 * HEADS, BN, 8), lambda i: (0, i, 0)),
        pl.BlockSpec((3, HID), lambda i: (0, 0)),
    ]


def _combine(num, den, exs, hh, bngb):
    return pl.pallas_call(
        _combine_body,
        grid=(NB,),
        in_specs=_combine_in_specs(),
        out_specs=pl.BlockSpec((BN, F_IN), lambda i: (i, 0)),
        out_shape=jax.ShapeDtypeStruct((N, F_IN), jnp.float32),
    )(num, den, exs, hh, bngb)


def _logits_body(h_ref, pw1_ref, pb1_ref, pw2_ref, pb2_ref, frag_ref,
                 lg_ref, bm_ref):
    h3 = h_ref[...][:, :HID]
    t = jnp.maximum(_dot(h3, pw1_ref[...], ((1,), (1,))) + pb1_ref[0][None, :],
                    0.0)
    lg = _dot(t, pw2_ref[...], ((1,), (1,)))
    lgv = lg[:, 0] + pb2_ref[0, 0]
    lg_ref[...] = lgv.reshape(1, 1, BN)
    labb = frag_ref[0, 0, :]
    m0 = jnp.max(jnp.where(labb == 0, lgv, -1e30))
    m1 = jnp.max(jnp.where(labb == 1, lgv, -1e30))
    lane = lax.broadcasted_iota(jnp.int32, (1, 1, 128), 2)
    bm_ref[...] = jnp.where(lane == 0, m0, jnp.where(lane == 1, m1, -1e30))


def _logits(hpad, pw1, pb1, pw2, pb2, frag3d):
    return pl.pallas_call(
        _logits_body,
        grid=(NB,),
        in_specs=[
            pl.BlockSpec((BN, F_IN), lambda i: (i, 0)),
            pl.BlockSpec((HID // 2, HID), lambda i: (0, 0)),
            pl.BlockSpec((1, HID // 2), lambda i: (0, 0)),
            pl.BlockSpec((8, HID // 2), lambda i: (0, 0)),
            pl.BlockSpec((1, 1), lambda i: (0, 0)),
            pl.BlockSpec((1, 1, BN), lambda i: (i, 0, 0)),
        ],
        out_specs=[
            pl.BlockSpec((1, 1, BN), lambda i: (i, 0, 0)),
            pl.BlockSpec((1, 1, 128), lambda i: (i, 0, 0)),
        ],
        out_shape=[
            jax.ShapeDtypeStruct((NB, 1, BN), jnp.float32),
            jax.ShapeDtypeStruct((NB, 1, 128), jnp.float32),
        ],
    )(hpad, pw1, pb1, pw2, pb2, frag3d)


def _pool_body(h_ref, lg_ref, batch_ref, frag_ref, bm_ref, f1_ref, f2_ref,
               F_acc, s_acc):
    i = pl.program_id(0)

    @pl.when(i == 0)
    def _init():
        F_acc[...] = jnp.zeros((2, G, HID), jnp.float32)
        s_acc[0] = 0.0
        s_acc[1] = 0.0

    mv = jnp.max(bm_ref[...], axis=(0, 1))
    m0 = mv[0]
    m1 = mv[1]
    lgv = lg_ref[0, 0, :]
    labb = frag_ref[0, 0, :]
    bb = batch_ref[0, 0, :]
    e0 = jnp.where(labb == 0, jnp.exp(lgv - m0), 0.0)
    e1 = jnp.where(labb == 1, jnp.exp(lgv - m1), 0.0)
    gid = lax.broadcasted_iota(jnp.int32, (BN, G), 1)
    oh = (bb[:, None] == gid).astype(jnp.float32)
    h3 = h_ref[...][:, :HID]
    A0 = oh * e0[:, None]
    A1 = oh * e1[:, None]
    F_acc[0] += _dot(A0, h3, ((0,), (0,)))
    F_acc[1] += _dot(A1, h3, ((0,), (0,)))
    s_acc[0] += jnp.sum(e0)
    s_acc[1] += jnp.sum(e1)
    f1_ref[...] = F_acc[0] / s_acc[0]
    f2_ref[...] = F_acc[1] / s_acc[1]


def _pool(hpad, lg3d, batch3d, frag3d, bm):
    return pl.pallas_call(
        _pool_body,
        grid=(NB,),
        in_specs=[
            pl.BlockSpec((BN, F_IN), lambda i: (i, 0)),
            pl.BlockSpec((1, 1, BN), lambda i: (i, 0, 0)),
            pl.BlockSpec((1, 1, BN), lambda i: (i, 0, 0)),
            pl.BlockSpec((1, 1, BN), lambda i: (i, 0, 0)),
            pl.BlockSpec((NB, 1, 128), lambda i: (0, 0, 0)),
        ],
        out_specs=[
            pl.BlockSpec((G, HID), lambda i: (0, 0)),
            pl.BlockSpec((G, HID), lambda i: (0, 0)),
        ],
        out_shape=[
            jax.ShapeDtypeStruct((G, HID), jnp.float32),
            jax.ShapeDtypeStruct((G, HID), jnp.float32),
        ],
        scratch_shapes=[
            pltpu.VMEM((2, G, HID), jnp.float32),
            pltpu.SMEM((2,), jnp.float32),
        ],
        compiler_params=pltpu.CompilerParams(
            dimension_semantics=("arbitrary",)),
    )(hpad, lg3d, batch3d, frag3d, bm)


def _heads_body(f1_ref, f2_ref, iw1_ref, ib1_ref, iw2_ref, ib2_ref,
                cw1_ref, cb1_ref, cw2_ref, cb2_ref, ln_ref,
                sw1_ref, sb1_ref, sw2_ref, sb2_ref, hw_ref, hb_ref,
                c1_ref, c2_ref, sm_ref):
    f1 = f1_ref[...]
    f2 = f2_ref[...]

    def contrast(f):
        t = jnp.maximum(_dot(f, cw1_ref[...], ((1,), (1,))) + cb1_ref[0][None, :], 0.0)
        p = _dot(t, cw2_ref[...], ((1,), (1,))) + cb2_ref[0][None, :]
        mu = jnp.mean(p, axis=1, keepdims=True)
        var = jnp.mean((p - mu) ** 2, axis=1, keepdims=True)
        p = (p - mu) / jnp.sqrt(var + 1e-5) * ln_ref[0][None, :] + ln_ref[1][None, :]
        nrm = jnp.sqrt(jnp.sum(p * p, axis=1, keepdims=True))
        return p / jnp.maximum(nrm, 1e-12)

    c1_ref[...] = contrast(f1)
    c2_ref[...] = contrast(f2)

    comb = jnp.concatenate([f1, f2], axis=1)
    t = jnp.maximum(_dot(comb, iw1_ref[...], ((1,), (1,))) + ib1_ref[0][None, :], 0.0)
    inter = jnp.maximum(_dot(t, iw2_ref[...], ((1,), (1,))) + ib2_ref[0][None, :], 0.0)
    t = jnp.maximum(_dot(inter, sw1_ref[...], ((1,), (1,))) + sb1_ref[0][None, :], 0.0)
    sh = jnp.maximum(_dot(t, sw2_ref[...], ((1,), (1,))) + sb2_ref[0][None, :], 0.0)
    v = _dot(sh, hw_ref[...], ((1,), (1,))) + hb_ref[0][None, :]
    lane = lax.broadcasted_iota(jnp.int32, (G, 8), 1)
    sm_ref[...] = jnp.where(lane == 1, jax.nn.sigmoid(v), v)


def _heads(f1, f2, pp):
    ins = [f1, f2, pp['iw1'], pp['ib1'], pp['iw2'], pp['ib2'],
           pp['cw1'], pp['cb1'], pp['cw2'], pp['cb2'], pp['ln'],
           pp['sw1'], pp['sb1'], pp['sw2'], pp['sb2'], pp['hw'], pp['hb']]
    return pl.pallas_call(
        _heads_body,
        in_specs=[pl.BlockSpec(a.shape, lambda: tuple(0 for _ in a.shape))
                  for a in ins],
        out_specs=[
            pl.BlockSpec((G, EMB), lambda: (0, 0)),
            pl.BlockSpec((G, EMB), lambda: (0, 0)),
            pl.BlockSpec((G, 8), lambda: (0, 0)),
        ],
        out_shape=[
            jax.ShapeDtypeStruct((G, EMB), jnp.float32),
            jax.ShapeDtypeStruct((G, EMB), jnp.float32),
            jax.ShapeDtypeStruct((G, 8), jnp.float32),
        ],
    )(*ins)


def _edge_body(hh_ref, aS_ref, aD_ref, src_ref, dst_ref, zr_ref,
               zd_ref, num_ref, den_ref,
               colS, colD, srcv0, srcv1, dstv0, dstv1,
               exv0, exv1, rows0, rows1, num_acc, den_acc,
               semG0, semG1, semI0, semI1):
    srcvs = [srcv0, srcv1]
    dstvs = [dstv0, dstv1]
    exvs = [exv0, exv1]
    rowss = [rows0, rows1]
    semGs = [semG0, semG1]
    semIs = [semI0, semI1]


    c = lax.axis_index("c")
    s = lax.axis_index("s")

    nzr = NZ - (NZ // 16) * 16
    nz = jnp.where(s < nzr, NZ // 16 + 1, NZ // 16)
    ncr = NCHUNK - (NCHUNK // 16) * 16
    nchunks = jnp.where(s < ncr, NCHUNK // 16 + 1, NCHUNK // 16)

    iota = lax.iota(jnp.int32, 16)
    rowoff = iota // 8
    coloff = iota % 8

    def phase_body(p, _):
        hs = p // 2
        half = p % 2
        h = 2 * hs + c
        t = 2 * h + half

        def zero_body(j, _):
            base = (s + 16 * j) * ZCH
            pltpu.sync_copy(zr_ref, num_acc.at[pl.ds(base, ZCH)])

            @pl.when(half == 0)
            def _():
                pltpu.sync_copy(zd_ref, den_acc.at[pl.ds(base, ZCH)])
            return 0

        lax.fori_loop(0, nz, zero_body, 0)

        @pl.when(half == 0)
        def _():
            def col_body(j, _):
                pltpu.sync_copy(aS_ref.at[j, h], colS.at[pl.ds(j * BN, BN)])
                pltpu.sync_copy(aD_ref.at[j, h], colD.at[pl.ds(j * BN, BN)])
                return 0

            lax.fori_loop(0, NB, col_body, 0)

        plsc.subcore_barrier()

        def issue_idx(k, b):
            eb = (s + k * 16) * CH
            pltpu.async_copy(src_ref.at[pl.ds(eb, CH)], srcvs[b], semIs[b])
            pltpu.async_copy(dst_ref.at[pl.ds(eb, CH)], dstvs[b], semIs[b])

        def wait_idx(k, b):
            eb = (s + k * 16) * CH
            pltpu.make_async_copy(src_ref.at[pl.ds(eb, CH)], srcvs[b],
                                  semIs[b]).wait()
            pltpu.make_async_copy(dst_ref.at[pl.ds(eb, CH)], dstvs[b],
                                  semIs[b]).wait()

        issue_idx(0, 0)

        def group_body(g, _):
            for b in range(2):
                tt = g * 2 + b
                p = 1 - b
                nx = 1 - b

                @pl.when(tt <= nchunks - 1)
                def _():
                    wait_idx(tt, b)
                    for i in range(CH // 16):
                        sidx = srcvs[b][pl.ds(i * 16, 16)]
                        didx = dstvs[b][pl.ds(i * 16, 16)]
                        av = plsc.load_gather(colS, [sidx])
                        bv = plsc.load_gather(colD, [didx])
                        al = av + bv
                        al = jnp.where(al > 0, al, 0.2 * al)
                        exvs[b][pl.ds(i * 16, 16)] = jnp.exp(al)
                        srcvs[b][pl.ds(i * 16, 16)] = sidx + t * N
                    pltpu.async_copy(hh_ref.at[srcvs[b]], rowss[b], semGs[b])

                @pl.when((tt >= 1) & (tt <= nchunks))
                def _():
                    pltpu.make_async_copy(hh_ref.at[srcvs[p]], rowss[p],
                                          semGs[p]).wait()
                    for i in range(CH // 16):
                        ev = exvs[p][pl.ds(i * 16, 16)]
                        for gg in range(8):
                            e0 = ev[2 * gg]
                            e1 = ev[2 * gg + 1]
                            ridx = rowoff + (i * 16 + 2 * gg)
                            v = plsc.load_gather(rowss[p], [ridx, coloff])
                            sc = jnp.where(iota < 8, e0, e1)
                            plsc.store_scatter(rowss[p], [ridx, coloff],
                                               v * sc)
                    pltpu.sync_copy(rowss[p], num_acc.at[dstvs[p]], add=True)

                    @pl.when(half == 0)
                    def _():
                        pltpu.sync_copy(exvs[p], den_acc.at[dstvs[p]],
                                        add=True)

                @pl.when(tt + 1 <= nchunks - 1)
                def _():
                    issue_idx(tt + 1, nx)
            return 0

        ngroups = (nchunks + 2) // 2
        lax.fori_loop(0, ngroups, group_body, 0)
        plsc.subcore_barrier()

        def out_body(j, _):
            jb = s + 16 * j
            base = jb * ZCH
            pltpu.sync_copy(num_acc.at[pl.ds(base, ZCH)],
                            num_ref.at[pl.ds(t * N + base, ZCH)])

            @pl.when(half == 0)
            def _():
                pltpu.sync_copy(den_acc.at[pl.ds(base, ZCH)],
                                den_ref.at[jb, h])
            return 0

        lax.fori_loop(0, nz, out_body, 0)
        plsc.subcore_barrier()
        return 0

    lax.fori_loop(0, 4, phase_body, 0)


def _edge_pass(hh_flat, aS_flat, aD_flat, src, dst, zrows, zden):
    kern = pl.kernel(
        _edge_body,
        out_type=(
            jax.ShapeDtypeStruct((2 * HEADS * N, 8), jnp.float32),
            jax.ShapeDtypeStruct((NB, HEADS, BN), jnp.float32),
        ),
        mesh=_sc_mesh(),
        scratch_types=(
            [pltpu.VMEM((N,), jnp.float32)] * 2
            + [pltpu.VMEM((CH,), jnp.int32)] * 4
            + [pltpu.VMEM((CH,), jnp.float32)] * 2
            + [pltpu.VMEM((CH, 8), jnp.float32)] * 2
            + [pltpu.VMEM_SHARED((N, 8), jnp.float32),
               pltpu.VMEM_SHARED((N,), jnp.float32)]
            + [pltpu.SemaphoreType.DMA] * 4
        ),
        compiler_params=pltpu.CompilerParams(needs_layout_passes=False,
                                             use_tc_tiling_on_sc=False),
    )
    return kern(hh_flat, aS_flat, aD_flat, src, dst, zrows, zden)


def kernel(x, edge_index, batch, fragment_labels, params):
    src = edge_index[0]
    dst = edge_index[1]
    zrows = jnp.zeros((ZCH, 8), jnp.float32)
    zden = jnp.zeros((ZCH,), jnp.float32)

    gat = params['gat']
    bn = params['bn']

    def axp(a):
        eye = jnp.eye(HEADS, dtype=jnp.float32)
        return (eye[:, :, None] * a[:, None, :]).reshape(HEADS, HID)

    def padw(w):
        return jnp.concatenate(
            [w, jnp.zeros((HID, F_IN - w.shape[1]), jnp.float32)], axis=1) \
            if w.shape[1] < F_IN else w

    Wp = jnp.stack([padw(gat[l]['W']) for l in range(3)])
    asxp = jnp.stack([axp(gat[l]['as']) for l in range(3)])
    adxp = jnp.stack([axp(gat[l]['ad']) for l in range(3)])
    bngbp = jnp.stack([jnp.stack([gat[l]['b'], bn[l]['g'], bn[l]['b']])
                       for l in range(3)])
    def layer_body(h, xs):
        W, asx, adx, bngb = xs
        hh, aS, aD, exs = _prep(h, W, asx, adx)
        hf = hh.reshape(2 * HEADS * N, 8)
        num, den = _edge_pass(hf, aS, aD, src, dst, zrows, zden)
        num = num.reshape(2 * HEADS, N, 8)
        hnew = _combine(num, den, exs, hh, bngb)
        return hnew, None

    hpad, _ = lax.scan(layer_body, x, (Wp, asxp, adxp, bngbp))

    pool = params['pool']
    frag3d = fragment_labels.reshape(NB, 1, BN)
    batch3d = batch.reshape(NB, 1, BN)
    pw2p = jnp.concatenate(
        [pool['w2'], jnp.zeros((7, HID // 2), jnp.float32)], axis=0)
    lg3d, bm = _logits(hpad, pool['w1'], pool['b1'].reshape(1, -1),
                       pw2p, pool['b2'].reshape(1, 1), frag3d)
    f1, f2 = _pool(hpad, lg3d, batch3d, frag3d, bm)

    it = params['inter']
    cp = params['contr']
    mt = params['mt']
    hw = jnp.concatenate([mt['cw'], mt['rw'], mt['mw'],
                          jnp.zeros((4, HID // 8), jnp.float32)], axis=0)
    hb = jnp.concatenate([mt['cb'], mt['rb'], mt['mb'],
                          jnp.zeros((4,), jnp.float32)]).reshape(1, 8)
    pp = {
        'iw1': it['w1'], 'ib1': it['b1'].reshape(1, -1),
        'iw2': it['w2'], 'ib2': it['b2'].reshape(1, -1),
        'cw1': cp['w1'], 'cb1': cp['b1'].reshape(1, -1),
        'cw2': cp['w2'], 'cb2': cp['b2'].reshape(1, -1),
        'ln': jnp.stack([cp['lng'], cp['lnb']]),
        'sw1': mt['sw1'], 'sb1': mt['sb1'].reshape(1, -1),
        'sw2': mt['sw2'], 'sb2': mt['sb2'].reshape(1, -1),
        'hw': hw, 'hb': hb,
    }
    c1, c2, sm = _heads(f1, f2, pp)
    cls = sm[:, 0:1]
    reg = sm[:, 1:2]
    mc = sm[:, 2:4]
    return (c1, c2, cls, reg, mc)


# speedup vs baseline: 1.2896x; 1.0033x over previous
import functools

import jax
import jax.numpy as jnp
from jax import lax
from jax.experimental import pallas as pl
from jax.experimental.pallas import tpu as pltpu
from jax.experimental.pallas import tpu_sc as plsc

N = 50000
E = 800000
F_IN = 128
HID = 64
HEADS = 4
OUTC = 16
G = 64
EMB = 128

BN = 2000
NB = N // BN
CH = 128
NCHUNK = E // CH
ZCH = 2000
NZ = N // ZCH

_HI = jax.lax.Precision.HIGHEST


def _dot(a, b, dims):
    return lax.dot_general(a, b, (dims, ((), ())), precision=_HI,
                           preferred_element_type=jnp.float32)


@functools.cache
def _sc_mesh():
    return plsc.VectorSubcoreMesh(core_axis_name="c", subcore_axis_name="s",
                                  num_cores=2, num_subcores=16)


def _prep_body(x_ref, w_ref, asx_ref, adx_ref, hh_ref, aS_ref, aD_ref,
               ex_ref):
    hhfull = _dot(x_ref[...], w_ref[...], ((1,), (1,)))
    for t in range(2 * HEADS):
        hh_ref[t] = hhfull[:, t * 8:(t + 1) * 8]
    a_s = _dot(asx_ref[...], hhfull, ((1,), (1,)))
    a_d = _dot(adx_ref[...], hhfull, ((1,), (1,)))
    aS_ref[0] = a_s
    aD_ref[0] = a_d
    al = a_s + a_d
    ex_ref[0] = jnp.exp(jnp.where(al > 0, al, 0.2 * al))


def _prep(x, W, asx, adx):
    fin = x.shape[1]
    return pl.pallas_call(
        _prep_body,
        grid=(NB,),
        in_specs=[
            pl.BlockSpec((BN, fin), lambda i: (i, 0)),
            pl.BlockSpec((HID, fin), lambda i: (0, 0)),
            pl.BlockSpec((HEADS, HID), lambda i: (0, 0)),
            pl.BlockSpec((HEADS, HID), lambda i: (0, 0)),
        ],
        out_specs=[
            pl.BlockSpec((2 * HEADS, BN, 8), lambda i: (0, i, 0)),
            pl.BlockSpec((1, HEADS, BN), lambda i: (i, 0, 0)),
            pl.BlockSpec((1, HEADS, BN), lambda i: (i, 0, 0)),
            pl.BlockSpec((1, HEADS, BN), lambda i: (i, 0, 0)),
        ],
        out_shape=[
            jax.ShapeDtypeStruct((2 * HEADS, N, 8), jnp.float32),
            jax.ShapeDtypeStruct((NB, HEADS, BN), jnp.float32),
            jax.ShapeDtypeStruct((NB, HEADS, BN), jnp.float32),
            jax.ShapeDtypeStruct((NB, HEADS, BN), jnp.float32),
        ],
    )(x, W, asx, adx)


def _combine_cols(num_ref, den_ref, exs_ref, hh_ref, bngb_ref):
    cols = []
    for t in range(2 * HEADS):
        h = t // 2
        numh = num_ref[t]
        hhh = hh_ref[t]
        exh = exs_ref[0, h]
        dtot = den_ref[0, h] + exh + 1e-16
        cols.append((numh + exh[:, None] * hhh) / dtot[:, None])
    hnew = jnp.concatenate(cols, axis=1)
    bb = bngb_ref[0][None, :]
    gg = bngb_ref[1][None, :]
    be = bngb_ref[2][None, :]
    return jnp.maximum((hnew + bb) * gg + be, 0.0)


def _combine_body(num_ref, den_ref, exs_ref, hh_ref, bngb_ref, out_ref):
    out_ref[...] = _combine_cols(num_ref, den_ref, exs_ref, hh_ref, bngb_ref)


def _combine_in_specs():
    return [
        pl.BlockSpec((2 * HEADS, BN, 8), lambda i: (0, i, 0)),
        pl.BlockSpec((1, HEADS, BN), lambda i: (i, 0, 0)),
        pl.BlockSpec((1, HEADS, BN), lambda i: (i, 0, 0)),
        pl.BlockSpec((2 * HEADS, BN, 8), lambda i: (0, i, 0)),
        pl.BlockSpec((3, HID), lambda i: (0, 0)),
    ]


def _combine(num, den, exs, hh, bngb):
    return pl.pallas_call(
        _combine_body,
        grid=(NB,),
        in_specs=_combine_in_specs(),
        out_specs=pl.BlockSpec((BN, HID), lambda i: (i, 0)),
        out_shape=jax.ShapeDtypeStruct((N, HID), jnp.float32),
    )(num, den, exs, hh, bngb)


def _logits_body(h_ref, pw1_ref, pb1_ref, pw2_ref, pb2_ref, frag_ref,
                 lg_ref, bm_ref):
    h3 = h_ref[...]
    t = jnp.maximum(_dot(h3, pw1_ref[...], ((1,), (1,))) + pb1_ref[0][None, :],
                    0.0)
    lg = _dot(t, pw2_ref[...], ((1,), (1,)))
    lgv = lg[:, 0] + pb2_ref[0, 0]
    lg_ref[...] = lgv.reshape(1, 1, BN)
    labb = frag_ref[0, 0, :]
    m0 = jnp.max(jnp.where(labb == 0, lgv, -1e30))
    m1 = jnp.max(jnp.where(labb == 1, lgv, -1e30))
    lane = lax.broadcasted_iota(jnp.int32, (1, 1, 128), 2)
    bm_ref[...] = jnp.where(lane == 0, m0, jnp.where(lane == 1, m1, -1e30))


def _logits(hpad, pw1, pb1, pw2, pb2, frag3d):
    return pl.pallas_call(
        _logits_body,
        grid=(NB,),
        in_specs=[
            pl.BlockSpec((BN, HID), lambda i: (i, 0)),
            pl.BlockSpec((HID // 2, HID), lambda i: (0, 0)),
            pl.BlockSpec((1, HID // 2), lambda i: (0, 0)),
            pl.BlockSpec((8, HID // 2), lambda i: (0, 0)),
            pl.BlockSpec((1, 1), lambda i: (0, 0)),
            pl.BlockSpec((1, 1, BN), lambda i: (i, 0, 0)),
        ],
        out_specs=[
            pl.BlockSpec((1, 1, BN), lambda i: (i, 0, 0)),
            pl.BlockSpec((1, 1, 128), lambda i: (i, 0, 0)),
        ],
        out_shape=[
            jax.ShapeDtypeStruct((NB, 1, BN), jnp.float32),
            jax.ShapeDtypeStruct((NB, 1, 128), jnp.float32),
        ],
    )(hpad, pw1, pb1, pw2, pb2, frag3d)


def _pool_body(h_ref, lg_ref, batch_ref, frag_ref, bm_ref, f1_ref, f2_ref,
               F_acc, s_acc):
    i = pl.program_id(0)

    @pl.when(i == 0)
    def _init():
        F_acc[...] = jnp.zeros((2, G, HID), jnp.float32)
        s_acc[0] = 0.0
        s_acc[1] = 0.0

    mv = jnp.max(bm_ref[...], axis=(0, 1))
    m0 = mv[0]
    m1 = mv[1]
    lgv = lg_ref[0, 0, :]
    labb = frag_ref[0, 0, :]
    bb = batch_ref[0, 0, :]
    e0 = jnp.where(labb == 0, jnp.exp(lgv - m0), 0.0)
    e1 = jnp.where(labb == 1, jnp.exp(lgv - m1), 0.0)
    gid = lax.broadcasted_iota(jnp.int32, (BN, G), 1)
    oh = (bb[:, None] == gid).astype(jnp.float32)
    h3 = h_ref[...]
    A0 = oh * e0[:, None]
    A1 = oh * e1[:, None]
    F_acc[0] += _dot(A0, h3, ((0,), (0,)))
    F_acc[1] += _dot(A1, h3, ((0,), (0,)))
    s_acc[0] += jnp.sum(e0)
    s_acc[1] += jnp.sum(e1)
    f1_ref[...] = F_acc[0] / s_acc[0]
    f2_ref[...] = F_acc[1] / s_acc[1]


def _pool(hpad, lg3d, batch3d, frag3d, bm):
    return pl.pallas_call(
        _pool_body,
        grid=(NB,),
        in_specs=[
            pl.BlockSpec((BN, HID), lambda i: (i, 0)),
            pl.BlockSpec((1, 1, BN), lambda i: (i, 0, 0)),
            pl.BlockSpec((1, 1, BN), lambda i: (i, 0, 0)),
            pl.BlockSpec((1, 1, BN), lambda i: (i, 0, 0)),
            pl.BlockSpec((NB, 1, 128), lambda i: (0, 0, 0)),
        ],
        out_specs=[
            pl.BlockSpec((G, HID), lambda i: (0, 0)),
            pl.BlockSpec((G, HID), lambda i: (0, 0)),
        ],
        out_shape=[
            jax.ShapeDtypeStruct((G, HID), jnp.float32),
            jax.ShapeDtypeStruct((G, HID), jnp.float32),
        ],
        scratch_shapes=[
            pltpu.VMEM((2, G, HID), jnp.float32),
            pltpu.SMEM((2,), jnp.float32),
        ],
        compiler_params=pltpu.CompilerParams(
            dimension_semantics=("arbitrary",)),
    )(hpad, lg3d, batch3d, frag3d, bm)


def _heads_body(f1_ref, f2_ref, iw1_ref, ib1_ref, iw2_ref, ib2_ref,
                cw1_ref, cb1_ref, cw2_ref, cb2_ref, ln_ref,
                sw1_ref, sb1_ref, sw2_ref, sb2_ref, hw_ref, hb_ref,
                c1_ref, c2_ref, sm_ref):
    f1 = f1_ref[...]
    f2 = f2_ref[...]

    def contrast(f):
        t = jnp.maximum(_dot(f, cw1_ref[...], ((1,), (1,))) + cb1_ref[0][None, :], 0.0)
        p = _dot(t, cw2_ref[...], ((1,), (1,))) + cb2_ref[0][None, :]
        mu = jnp.mean(p, axis=1, keepdims=True)
        var = jnp.mean((p - mu) ** 2, axis=1, keepdims=True)
        p = (p - mu) / jnp.sqrt(var + 1e-5) * ln_ref[0][None, :] + ln_ref[1][None, :]
        nrm = jnp.sqrt(jnp.sum(p * p, axis=1, keepdims=True))
        return p / jnp.maximum(nrm, 1e-12)

    c1_ref[...] = contrast(f1)
    c2_ref[...] = contrast(f2)

    comb = jnp.concatenate([f1, f2], axis=1)
    t = jnp.maximum(_dot(comb, iw1_ref[...], ((1,), (1,))) + ib1_ref[0][None, :], 0.0)
    inter = jnp.maximum(_dot(t, iw2_ref[...], ((1,), (1,))) + ib2_ref[0][None, :], 0.0)
    t = jnp.maximum(_dot(inter, sw1_ref[...], ((1,), (1,))) + sb1_ref[0][None, :], 0.0)
    sh = jnp.maximum(_dot(t, sw2_ref[...], ((1,), (1,))) + sb2_ref[0][None, :], 0.0)
    v = _dot(sh, hw_ref[...], ((1,), (1,))) + hb_ref[0][None, :]
    lane = lax.broadcasted_iota(jnp.int32, (G, 8), 1)
    sm_ref[...] = jnp.where(lane == 1, jax.nn.sigmoid(v), v)


def _heads(f1, f2, pp):
    ins = [f1, f2, pp['iw1'], pp['ib1'], pp['iw2'], pp['ib2'],
           pp['cw1'], pp['cb1'], pp['cw2'], pp['cb2'], pp['ln'],
           pp['sw1'], pp['sb1'], pp['sw2'], pp['sb2'], pp['hw'], pp['hb']]
    return pl.pallas_call(
        _heads_body,
        in_specs=[pl.BlockSpec(a.shape, lambda: tuple(0 for _ in a.shape))
                  for a in ins],
        out_specs=[
            pl.BlockSpec((G, EMB), lambda: (0, 0)),
            pl.BlockSpec((G, EMB), lambda: (0, 0)),
            pl.BlockSpec((G, 8), lambda: (0, 0)),
        ],
        out_shape=[
            jax.ShapeDtypeStruct((G, EMB), jnp.float32),
            jax.ShapeDtypeStruct((G, EMB), jnp.float32),
            jax.ShapeDtypeStruct((G, 8), jnp.float32),
        ],
    )(*ins)


def _edge_body(hh_ref, aS_ref, aD_ref, src_ref, dst_ref, zr_ref,
               zd_ref, num_ref, den_ref,
               colS, colD, srcv0, srcv1, dstv0, dstv1,
               exv0, exv1, rows0, rows1, num_acc, den_acc,
               semG0, semG1, semI0, semI1):
    srcvs = [srcv0, srcv1]
    dstvs = [dstv0, dstv1]
    exvs = [exv0, exv1]
    rowss = [rows0, rows1]
    semGs = [semG0, semG1]
    semIs = [semI0, semI1]


    c = lax.axis_index("c")
    s = lax.axis_index("s")

    nzr = NZ - (NZ // 16) * 16
    nz = jnp.where(s < nzr, NZ // 16 + 1, NZ // 16)
    ncr = NCHUNK - (NCHUNK // 16) * 16
    nchunks = jnp.where(s < ncr, NCHUNK // 16 + 1, NCHUNK // 16)

    iota = lax.iota(jnp.int32, 16)
    rowoff = iota // 8
    coloff = iota % 8

    def phase_body(p, _):
        hs = p // 2
        half = p % 2
        h = 2 * hs + c
        t = 2 * h + half

        def zero_body(j, _):
            base = (s + 16 * j) * ZCH
            pltpu.sync_copy(zr_ref, num_acc.at[pl.ds(base, ZCH)])

            @pl.when(half == 0)
            def _():
                pltpu.sync_copy(zd_ref, den_acc.at[pl.ds(base, ZCH)])
            return 0

        lax.fori_loop(0, nz, zero_body, 0)

        @pl.when(half == 0)
        def _():
            def col_body(j, _):
                pltpu.sync_copy(aS_ref.at[j, h], colS.at[pl.ds(j * BN, BN)])
                pltpu.sync_copy(aD_ref.at[j, h], colD.at[pl.ds(j * BN, BN)])
                return 0

            lax.fori_loop(0, NB, col_body, 0)

        plsc.subcore_barrier()

        def issue_idx(k, b):
            eb = (s + k * 16) * CH
            pltpu.async_copy(src_ref.at[pl.ds(eb, CH)], srcvs[b], semIs[b])
            pltpu.async_copy(dst_ref.at[pl.ds(eb, CH)], dstvs[b], semIs[b])

        def wait_idx(k, b):
            eb = (s + k * 16) * CH
            pltpu.make_async_copy(src_ref.at[pl.ds(eb, CH)], srcvs[b],
                                  semIs[b]).wait()
            pltpu.make_async_copy(dst_ref.at[pl.ds(eb, CH)], dstvs[b],
                                  semIs[b]).wait()

        issue_idx(0, 0)

        def group_body(g, _):
            for b in range(2):
                tt = g * 2 + b
                p = 1 - b
                nx = 1 - b

                @pl.when(tt <= nchunks - 1)
                def _():
                    wait_idx(tt, b)
                    for i in range(CH // 16):
                        sidx = srcvs[b][pl.ds(i * 16, 16)]
                        didx = dstvs[b][pl.ds(i * 16, 16)]
                        av = plsc.load_gather(colS, [sidx])
                        bv = plsc.load_gather(colD, [didx])
                        al = av + bv
                        al = jnp.where(al > 0, al, 0.2 * al)
                        exvs[b][pl.ds(i * 16, 16)] = jnp.exp(al)
                        srcvs[b][pl.ds(i * 16, 16)] = sidx + t * N
                    pltpu.async_copy(hh_ref.at[srcvs[b]], rowss[b], semGs[b])

                @pl.when((tt >= 1) & (tt <= nchunks))
                def _():
                    pltpu.make_async_copy(hh_ref.at[srcvs[p]], rowss[p],
                                          semGs[p]).wait()
                    for i in range(CH // 16):
                        ev = exvs[p][pl.ds(i * 16, 16)]
                        for gg in range(8):
                            e0 = ev[2 * gg]
                            e1 = ev[2 * gg + 1]
                            ridx = rowoff + (i * 16 + 2 * gg)
                            v = plsc.load_gather(rowss[p], [ridx, coloff])
                            sc = jnp.where(iota < 8, e0, e1)
                            plsc.store_scatter(rowss[p], [ridx, coloff],
                                               v * sc)
                    pltpu.sync_copy(rowss[p], num_acc.at[dstvs[p]], add=True)

                    @pl.when(half == 0)
                    def _():
                        pltpu.sync_copy(exvs[p], den_acc.at[dstvs[p]],
                                        add=True)

                @pl.when(tt + 1 <= nchunks - 1)
                def _():
                    issue_idx(tt + 1, nx)
            return 0

        ngroups = (nchunks + 2) // 2
        lax.fori_loop(0, ngroups, group_body, 0)
        plsc.subcore_barrier()

        def out_body(j, _):
            jb = s + 16 * j
            base = jb * ZCH
            pltpu.sync_copy(num_acc.at[pl.ds(base, ZCH)],
                            num_ref.at[pl.ds(t * N + base, ZCH)])

            @pl.when(half == 0)
            def _():
                pltpu.sync_copy(den_acc.at[pl.ds(base, ZCH)],
                                den_ref.at[jb, h])
            return 0

        lax.fori_loop(0, nz, out_body, 0)
        plsc.subcore_barrier()
        return 0

    lax.fori_loop(0, 4, phase_body, 0)


def _edge_pass(hh_flat, aS_flat, aD_flat, src, dst, zrows, zden):
    kern = pl.kernel(
        _edge_body,
        out_type=(
            jax.ShapeDtypeStruct((2 * HEADS * N, 8), jnp.float32),
            jax.ShapeDtypeStruct((NB, HEADS, BN), jnp.float32),
        ),
        mesh=_sc_mesh(),
        scratch_types=(
            [pltpu.VMEM((N,), jnp.float32)] * 2
            + [pltpu.VMEM((CH,), jnp.int32)] * 4
            + [pltpu.VMEM((CH,), jnp.float32)] * 2
            + [pltpu.VMEM((CH, 8), jnp.float32)] * 2
            + [pltpu.VMEM_SHARED((N, 8), jnp.float32),
               pltpu.VMEM_SHARED((N,), jnp.float32)]
            + [pltpu.SemaphoreType.DMA] * 4
        ),
        compiler_params=pltpu.CompilerParams(needs_layout_passes=False,
                                             use_tc_tiling_on_sc=False),
    )
    return kern(hh_flat, aS_flat, aD_flat, src, dst, zrows, zden)


def kernel(x, edge_index, batch, fragment_labels, params):
    src = edge_index[0]
    dst = edge_index[1]
    zrows = jnp.zeros((ZCH, 8), jnp.float32)
    zden = jnp.zeros((ZCH,), jnp.float32)

    gat = params['gat']
    bn = params['bn']

    def axp(a):
        eye = jnp.eye(HEADS, dtype=jnp.float32)
        return (eye[:, :, None] * a[:, None, :]).reshape(HEADS, HID)

    Wp = jnp.stack([gat[l]['W'] for l in (1, 2)])
    asxp = jnp.stack([axp(gat[l]['as']) for l in (1, 2)])
    adxp = jnp.stack([axp(gat[l]['ad']) for l in (1, 2)])
    bngbp = jnp.stack([jnp.stack([gat[l]['b'], bn[l]['g'], bn[l]['b']])
                       for l in (1, 2)])

    def run_layer(h, W, asx, adx, bngb):
        hh, aS, aD, exs = _prep(h, W, asx, adx)
        hf = hh.reshape(2 * HEADS * N, 8)
        num, den = _edge_pass(hf, aS, aD, src, dst, zrows, zden)
        num = num.reshape(2 * HEADS, N, 8)
        return _combine(num, den, exs, hh, bngb)

    h1 = run_layer(x, gat[0]['W'], axp(gat[0]['as']), axp(gat[0]['ad']),
                   jnp.stack([gat[0]['b'], bn[0]['g'], bn[0]['b']]))

    def layer_body(h, xs):
        W, asx, adx, bngb = xs
        return run_layer(h, W, asx, adx, bngb), None

    hpad, _ = lax.scan(layer_body, h1, (Wp, asxp, adxp, bngbp))

    pool = params['pool']
    frag3d = fragment_labels.reshape(NB, 1, BN)
    batch3d = batch.reshape(NB, 1, BN)
    pw2p = jnp.concatenate(
        [pool['w2'], jnp.zeros((7, HID // 2), jnp.float32)], axis=0)
    lg3d, bm = _logits(hpad, pool['w1'], pool['b1'].reshape(1, -1),
                       pw2p, pool['b2'].reshape(1, 1), frag3d)
    f1, f2 = _pool(hpad, lg3d, batch3d, frag3d, bm)

    it = params['inter']
    cp = params['contr']
    mt = params['mt']
    hw = jnp.concatenate([mt['cw'], mt['rw'], mt['mw'],
                          jnp.zeros((4, HID // 8), jnp.float32)], axis=0)
    hb = jnp.concatenate([mt['cb'], mt['rb'], mt['mb'],
                          jnp.zeros((4,), jnp.float32)]).reshape(1, 8)
    pp = {
        'iw1': it['w1'], 'ib1': it['b1'].reshape(1, -1),
        'iw2': it['w2'], 'ib2': it['b2'].reshape(1, -1),
        'cw1': cp['w1'], 'cb1': cp['b1'].reshape(1, -1),
        'cw2': cp['w2'], 'cb2': cp['b2'].reshape(1, -1),
        'ln': jnp.stack([cp['lng'], cp['lnb']]),
        'sw1': mt['sw1'], 'sb1': mt['sb1'].reshape(1, -1),
        'sw2': mt['sw2'], 'sb2': mt['sb2'].reshape(1, -1),
        'hw': hw, 'hb': hb,
    }
    c1, c2, sm = _heads(f1, f2, pp)
    cls = sm[:, 0:1]
    reg = sm[:, 1:2]
    mc = sm[:, 2:4]
    return (c1, c2, cls, reg, mc)
